# Initial kernel scaffold; baseline (speedup 1.0000x reference)
import jax
import jax.numpy as jnp
from jax.experimental import pallas as pl


def kernel(x, pos_table):
    raise NotImplementedError("write your pallas kernel here")


# speedup vs baseline: 1.7207x; 1.7207x over previous
import functools
import math

import jax
import jax.numpy as jnp
from jax.experimental import pallas as pl


def _pe_block(x_ref, pos_ref, o_ref, *, scale):
    o_ref[...] = x_ref[...] * scale + pos_ref[...][None, :, :]


def kernel(x, pos_table):
    batch, seq_len, d_model = x.shape
    scale = math.sqrt(d_model)
    bs = 256
    grid = (seq_len // bs,)
    return pl.pallas_call(
        functools.partial(_pe_block, scale=scale),
        grid=grid,
        in_specs=[
            pl.BlockSpec((batch, bs, d_model), lambda i: (0, i, 0)),
            pl.BlockSpec((bs, d_model), lambda i: (i, 0)),
        ],
        out_specs=pl.BlockSpec((batch, bs, d_model), lambda i: (0, i, 0)),
        out_shape=jax.ShapeDtypeStruct((batch, seq_len, d_model), x.dtype),
    )(x, pos_table[:seq_len])
